# Initial kernel scaffold; baseline (speedup 1.0000x reference)
import jax
import jax.numpy as jnp
from jax.experimental import pallas as pl


def kernel(x, x_real, edge_index, fc_W, fc_b, l1_W, l1_b, l2_W, l2_b, eps, convA_W, convA_b, convB_W, convB_b, convC_W, convC_b, convD_W, convD_b):
    raise NotImplementedError("write your pallas kernel here")


# speedup vs baseline: 6.4656x; 6.4656x over previous
import functools

import jax
import jax.numpy as jnp
from jax import lax
from jax.experimental import pallas as pl
from jax.experimental.pallas import tpu as pltpu
from jax.experimental.pallas import tpu_sc as plsc

N_NODES = 50000
N_EDGES = 800000
NC, NS = 2, 16
EPT = 51200
E_PAD = EPT * NS
W_EDGES = 512
NJ = W_EDGES // 128
NWIN = EPT // W_EDGES
ACC_ROWS = 50048
BN = 1000


def _leaky(v):
    return jnp.where(v >= 0, v, 0.01 * v)


def _fc_body(xr_ref, wT_ref, b_ref, o_ref):
    y = jnp.dot(xr_ref[...], wT_ref[...], preferred_element_type=jnp.float32)
    y = _leaky(y + b_ref[...])
    o_ref[0] = y[:, :32]
    o_ref[1] = y[:, 32:]


def _run_fc(x_real, fc_W, fc_b):
    grid = N_NODES // BN
    return pl.pallas_call(
        _fc_body,
        grid=(grid,),
        in_specs=[
            pl.BlockSpec((BN, 32), lambda i: (i, 0)),
            pl.BlockSpec((32, 64), lambda i: (0, 0)),
            pl.BlockSpec((1, 64), lambda i: (0, 0)),
        ],
        out_specs=pl.BlockSpec((2, BN, 32), lambda i: (0, i, 0)),
        out_shape=jax.ShapeDtypeStruct((2, N_NODES, 32), jnp.float32),
    )(x_real, fc_W.T, fc_b.reshape(1, 64))


@functools.cache
def _make_sc_segment_sum():
    mesh = plsc.VectorSubcoreMesh(
        core_axis_name="c", subcore_axis_name="s",
        num_cores=NC, num_subcores=NS)
    return pl.kernel(
        _sc_segment_sum_body,
        out_type=jax.ShapeDtypeStruct((NC, ACC_ROWS, 32), jnp.float32),
        mesh=mesh,
        scratch_types=[
            pltpu.VMEM((NJ, 128), jnp.int32),
            pltpu.VMEM((NJ, 128), jnp.int32),
            pltpu.VMEM((W_EDGES, 32), jnp.float32),
            pltpu.VMEM_SHARED((ACC_ROWS, 32), jnp.float32),
            pltpu.SemaphoreType.DMA,
        ],
        compiler_params=pltpu.CompilerParams(use_tc_tiling_on_sc=False),
    )


def _sc_segment_sum_body(src2d_hbm, dst2d_hbm, x1s_hbm, out_hbm,
                         idx_src, idx_dst, rows, acc, sem):
    c = lax.axis_index("c")
    s = lax.axis_index("s")

    def zbody(i, carry):
        rows[i, pl.ds(0, 16)] = jnp.zeros((16,), jnp.float32)
        rows[i, pl.ds(16, 16)] = jnp.zeros((16,), jnp.float32)
        return carry

    lax.fori_loop(0, W_EDGES, zbody, 0)
    zrows = ACC_ROWS // NS
    zbase = s * zrows
    for r in range(0, zrows, W_EDGES):
        size = min(W_EDGES, zrows - r)
        pltpu.sync_copy(rows.at[pl.ds(0, size)],
                        acc.at[pl.ds(zbase + r, size)])
    plsc.subcore_barrier()

    ept128 = EPT // 128
    src_base = c * (E_PAD // 128) + s * ept128
    dst_base = s * ept128

    def window(w, carry):
        row0 = w * NJ
        pltpu.sync_copy(src2d_hbm.at[pl.ds(src_base + row0, NJ)], idx_src)
        pltpu.sync_copy(dst2d_hbm.at[pl.ds(dst_base + row0, NJ)], idx_dst)
        descs = []
        for j in range(NJ):
            descs.append(pltpu.async_copy(
                x1s_hbm.at[idx_src.at[j]],
                rows.at[pl.ds(j * 128, 128)], sem))
        for d in descs:
            d.wait()
        for j in range(NJ):
            pltpu.sync_copy(rows.at[pl.ds(j * 128, 128)],
                            acc.at[idx_dst.at[j]], add=True)
        return carry

    lax.fori_loop(0, NWIN, window, 0)
    plsc.subcore_barrier()

    orows = ACC_ROWS // NS
    pltpu.sync_copy(acc.at[pl.ds(s * orows, orows)],
                    out_hbm.at[c, pl.ds(s * orows, orows)])


def _tail_body(x1_ref, agg_ref, xf_ref, eps_ref,
               l1T_ref, l1b_ref, l2T_ref, l2b_ref,
               sx_ref, sh_ref, wa_ref, ba_ref, wb_ref, bb_ref,
               wc_ref, bc_ref, wd_ref, bd_ref, o_ref):
    f32 = jnp.float32
    x1 = jnp.concatenate([x1_ref[0], x1_ref[1]], axis=1)
    agg = jnp.concatenate([agg_ref[0], agg_ref[1]], axis=1)
    h = eps_ref[0, 0] * x1 + agg
    h = _leaky(jnp.dot(h, l1T_ref[...], preferred_element_type=f32)
               + l1b_ref[...])
    h = _leaky(jnp.dot(h, l2T_ref[...], preferred_element_type=f32)
               + l2b_ref[...])
    h = _leaky(h)
    z = (jnp.dot(xf_ref[...], sx_ref[...], preferred_element_type=f32)
         + jnp.dot(h, sh_ref[...], preferred_element_type=f32))
    ya = _leaky(jnp.dot(z, wa_ref[...], preferred_element_type=f32)
                + ba_ref[...])
    yb = _leaky(jnp.dot(ya, wb_ref[...], preferred_element_type=f32)
                + bb_ref[...])
    yc = _leaky(jnp.dot(yb, wc_ref[...], preferred_element_type=f32)
                + bc_ref[...])
    o_ref[...] = _leaky(jnp.dot(yc, wd_ref[...], preferred_element_type=f32)
                        + bd_ref[...])


def _run_tail(x1s, agg, xflat, epsp1, l1T, l1b, l2T, l2b,
              Sx, Sh, WA, bA, WB, bB, WC, bC, WD, bD):
    grid = N_NODES // BN

    def whole(shape):
        return pl.BlockSpec(shape, lambda i: tuple(0 for _ in shape))

    return pl.pallas_call(
        _tail_body,
        grid=(grid,),
        in_specs=[
            pl.BlockSpec((2, BN, 32), lambda i: (0, i, 0)),
            pl.BlockSpec((2, BN, 32), lambda i: (0, i, 0)),
            pl.BlockSpec((BN, 16), lambda i: (i, 0)),
            whole((1, 1)),
            whole((64, 32)), whole((1, 32)),
            whole((32, 16)), whole((1, 16)),
            whole((16, 32)), whole((16, 32)),
            whole((32, 256)), whole((1, 256)),
            whole((256, 256)), whole((1, 256)),
            whole((256, 256)), whole((1, 256)),
            whole((256, 64)), whole((1, 64)),
        ],
        out_specs=pl.BlockSpec((BN, 64), lambda i: (i, 0)),
        out_shape=jax.ShapeDtypeStruct((N_NODES, 64), jnp.float32),
    )(x1s, agg, xflat, epsp1, l1T, l1b, l2T, l2b,
      Sx, Sh, WA, bA, WB, bB, WC, bC, WD, bD)


def kernel(x, x_real, edge_index, fc_W, fc_b, l1_W, l1_b, l2_W, l2_b, eps,
           convA_W, convA_b, convB_W, convB_b, convC_W, convC_b,
           convD_W, convD_b):
    f32 = jnp.float32

    src = edge_index[0]
    dst = edge_index[1]
    npad = E_PAD - N_EDGES
    ppos = jnp.arange(npad, dtype=jnp.int32)
    src_full = jnp.concatenate([src, ppos % N_NODES])
    dst_full = jnp.concatenate([dst, N_NODES + (ppos % 16)])
    src2d = jnp.stack([src_full, src_full + N_NODES]).reshape(
        2 * E_PAD // 128, 128)
    dst2d = dst_full.reshape(E_PAD // 128, 128)

    x1s = _run_fc(x_real, fc_W, fc_b)

    agg = _make_sc_segment_sum()(
        src2d, dst2d, x1s.reshape(2 * N_NODES, 32))

    l1T = l1_W.T
    l2T = l2_W.T
    ar16 = jnp.arange(16)
    colsx = 8 * (ar16 // 4) + (ar16 % 4)
    Sx = jnp.zeros((16, 32), f32).at[ar16, colsx].set(1.0)
    Sh = jnp.zeros((16, 32), f32).at[ar16, colsx + 4].set(1.0)
    WAf = convA_W.reshape(64, 8)
    WA = jnp.einsum('pq,oj->pjqo', jnp.eye(4, dtype=f32), WAf).reshape(32, 256)
    bA = jnp.tile(convA_b, 4).reshape(1, 256)
    WBf = convB_W[:, :, :, 0].transpose(0, 2, 1).reshape(128, 128)
    WB = jnp.einsum('pq,oj->pjqo', jnp.eye(2, dtype=f32), WBf).reshape(256, 256)
    bB = jnp.tile(convB_b, 2).reshape(1, 256)
    WC = convC_W[:, :, :, 0].transpose(0, 2, 1).reshape(256, 256).T
    bC = convC_b.reshape(1, 256)
    WD = convD_W.reshape(64, 256).T
    bD = convD_b.reshape(1, 64)

    xflat = x.reshape(N_NODES, 16)
    epsp1 = (1.0 + eps).reshape(1, 1)
    out = _run_tail(x1s, agg, xflat, epsp1,
                    l1T, l1_b.reshape(1, 32), l2T, l2_b.reshape(1, 16),
                    Sx, Sh, WA, bA, WB, bB, WC, bC, WD, bD)
    return out.reshape(N_NODES, 64, 1, 1)


# speedup vs baseline: 7.8757x; 1.2181x over previous
import functools

import jax
import jax.numpy as jnp
from jax import lax
from jax.experimental import pallas as pl
from jax.experimental.pallas import tpu as pltpu
from jax.experimental.pallas import tpu_sc as plsc

N_NODES = 50000
N_EDGES = 800000
NC, NS = 2, 16
EPT = 51200
E_PAD = EPT * NS
W_EDGES = 256
SUPER = 4
NSB = EPT // (SUPER * W_EDGES)
NJ = W_EDGES // 128
NWIN = EPT // W_EDGES
ACC_ROWS = 50048
BN = 1000


def _leaky(v):
    return jnp.where(v >= 0, v, 0.01 * v)


def _fc_body(xr_ref, wT_ref, b_ref, o_ref):
    y = jnp.dot(xr_ref[...], wT_ref[...], preferred_element_type=jnp.float32)
    y = _leaky(y + b_ref[...])
    o_ref[0] = y[:, :32]
    o_ref[1] = y[:, 32:]


def _run_fc(x_real, fc_W, fc_b):
    grid = N_NODES // BN
    return pl.pallas_call(
        _fc_body,
        grid=(grid,),
        in_specs=[
            pl.BlockSpec((BN, 32), lambda i: (i, 0)),
            pl.BlockSpec((32, 64), lambda i: (0, 0)),
            pl.BlockSpec((1, 64), lambda i: (0, 0)),
        ],
        out_specs=pl.BlockSpec((2, BN, 32), lambda i: (0, i, 0)),
        out_shape=jax.ShapeDtypeStruct((2, N_NODES, 32), jnp.float32),
    )(x_real, fc_W.T, fc_b.reshape(1, 64))


@functools.cache
def _make_sc_segment_sum():
    mesh = plsc.VectorSubcoreMesh(
        core_axis_name="c", subcore_axis_name="s",
        num_cores=NC, num_subcores=NS)
    return pl.kernel(
        _sc_segment_sum_body,
        out_type=jax.ShapeDtypeStruct((NC, ACC_ROWS, 32), jnp.float32),
        mesh=mesh,
        scratch_types=[
            pltpu.VMEM((2, SUPER * NJ, 128), jnp.int32),
            pltpu.VMEM((2, SUPER * NJ, 128), jnp.int32),
            pltpu.VMEM((2, W_EDGES, 32), jnp.float32),
            pltpu.VMEM_SHARED((ACC_ROWS, 32), jnp.float32),
            pltpu.SemaphoreType.DMA,
            pltpu.SemaphoreType.DMA,
            pltpu.SemaphoreType.DMA,
            pltpu.SemaphoreType.DMA,
        ],
        compiler_params=pltpu.CompilerParams(use_tc_tiling_on_sc=False),
    )


def _sc_segment_sum_body(src2d_hbm, dst2d_hbm, x1s_hbm, out_hbm,
                         idx_src, idx_dst, rows, acc,
                         sem_g, sem_s0, sem_s1, sem_i):
    c = lax.axis_index("c")
    s = lax.axis_index("s")
    sem_s = (sem_s0, sem_s1)

    def zbody(i, carry):
        rows[0, i, pl.ds(0, 16)] = jnp.zeros((16,), jnp.float32)
        rows[0, i, pl.ds(16, 16)] = jnp.zeros((16,), jnp.float32)
        return carry

    lax.fori_loop(0, W_EDGES, zbody, 0)
    zrows = ACC_ROWS // NS
    zbase = s * zrows
    for r in range(0, zrows, W_EDGES):
        size = min(W_EDGES, zrows - r)
        pltpu.sync_copy(rows.at[0, pl.ds(0, size)],
                        acc.at[pl.ds(zbase + r, size)])
    plsc.subcore_barrier()

    ept128 = EPT // 128
    src_base = c * (E_PAD // 128) + s * ept128
    dst_base = s * ept128
    SBR = SUPER * NJ

    pltpu.sync_copy(src2d_hbm.at[pl.ds(src_base, SBR)], idx_src.at[0])
    pltpu.sync_copy(dst2d_hbm.at[pl.ds(dst_base, SBR)], idx_dst.at[0])

    def superblock(sb, carry):
        ib = sb % 2
        nxt = 1 - ib

        @pl.when(sb > 0)
        def _():
            pltpu.make_async_copy(
                src2d_hbm.at[pl.ds(src_base, SBR)], idx_src.at[ib],
                sem_i).wait()
            pltpu.make_async_copy(
                dst2d_hbm.at[pl.ds(dst_base, SBR)], idx_dst.at[ib],
                sem_i).wait()

        for k in range(SUPER):
            b = k % 2
            w = sb * SUPER + k

            def drain(b=b):
                for j in range(NJ):
                    pltpu.make_async_copy(
                        rows.at[b, pl.ds(j * 128, 128)],
                        acc.at[idx_dst.at[ib, j]],
                        sem_s[b]).wait()

            if k < 2:
                pl.when(sb > 0)(drain)
            else:
                drain()

            gd = []
            for j in range(NJ):
                gd.append(pltpu.async_copy(
                    x1s_hbm.at[idx_src.at[ib, k * NJ + j]],
                    rows.at[b, pl.ds(j * 128, 128)], sem_g))

            if k == 1:
                @pl.when(sb + 1 < NSB)
                def _():
                    r0 = (sb + 1) * SBR
                    pltpu.async_copy(
                        src2d_hbm.at[pl.ds(src_base + r0, SBR)],
                        idx_src.at[nxt], sem_i)
                    pltpu.async_copy(
                        dst2d_hbm.at[pl.ds(dst_base + r0, SBR)],
                        idx_dst.at[nxt], sem_i)

            for d in gd:
                d.wait()

            for j in range(NJ):
                pltpu.async_copy(
                    rows.at[b, pl.ds(j * 128, 128)],
                    acc.at[idx_dst.at[ib, k * NJ + j]],
                    sem_s[b], add=True)
        return carry

    lax.fori_loop(0, NSB, superblock, 0)

    for b in range(2):
        for j in range(NJ):
            pltpu.make_async_copy(
                rows.at[b, pl.ds(j * 128, 128)],
                acc.at[idx_dst.at[0, j]],
                sem_s[b]).wait()
    plsc.subcore_barrier()

    orows = ACC_ROWS // NS
    pltpu.sync_copy(acc.at[pl.ds(s * orows, orows)],
                    out_hbm.at[c, pl.ds(s * orows, orows)])


def _tail_body(x1_ref, agg_ref, xf_ref, eps_ref,
               l1T_ref, l1b_ref, l2T_ref, l2b_ref,
               sx_ref, sh_ref, wa_ref, ba_ref, wb_ref, bb_ref,
               wc_ref, bc_ref, wd_ref, bd_ref, o_ref):
    f32 = jnp.float32
    x1 = jnp.concatenate([x1_ref[0], x1_ref[1]], axis=1)
    agg = jnp.concatenate([agg_ref[0], agg_ref[1]], axis=1)
    h = eps_ref[0, 0] * x1 + agg
    h = _leaky(jnp.dot(h, l1T_ref[...], preferred_element_type=f32)
               + l1b_ref[...])
    h = _leaky(jnp.dot(h, l2T_ref[...], preferred_element_type=f32)
               + l2b_ref[...])
    h = _leaky(h)
    z = (jnp.dot(xf_ref[...], sx_ref[...], preferred_element_type=f32)
         + jnp.dot(h, sh_ref[...], preferred_element_type=f32))
    ya = _leaky(jnp.dot(z, wa_ref[...], preferred_element_type=f32)
                + ba_ref[...])
    yb = _leaky(jnp.dot(ya, wb_ref[...], preferred_element_type=f32)
                + bb_ref[...])
    yc = _leaky(jnp.dot(yb, wc_ref[...], preferred_element_type=f32)
                + bc_ref[...])
    o_ref[...] = _leaky(jnp.dot(yc, wd_ref[...], preferred_element_type=f32)
                        + bd_ref[...])


def _run_tail(x1s, agg, xflat, epsp1, l1T, l1b, l2T, l2b,
              Sx, Sh, WA, bA, WB, bB, WC, bC, WD, bD):
    grid = N_NODES // BN

    def whole(shape):
        return pl.BlockSpec(shape, lambda i: tuple(0 for _ in shape))

    return pl.pallas_call(
        _tail_body,
        grid=(grid,),
        in_specs=[
            pl.BlockSpec((2, BN, 32), lambda i: (0, i, 0)),
            pl.BlockSpec((2, BN, 32), lambda i: (0, i, 0)),
            pl.BlockSpec((BN, 16), lambda i: (i, 0)),
            whole((1, 1)),
            whole((64, 32)), whole((1, 32)),
            whole((32, 16)), whole((1, 16)),
            whole((16, 32)), whole((16, 32)),
            whole((32, 256)), whole((1, 256)),
            whole((256, 256)), whole((1, 256)),
            whole((256, 256)), whole((1, 256)),
            whole((256, 64)), whole((1, 64)),
        ],
        out_specs=pl.BlockSpec((BN, 64), lambda i: (i, 0)),
        out_shape=jax.ShapeDtypeStruct((N_NODES, 64), jnp.float32),
    )(x1s, agg, xflat, epsp1, l1T, l1b, l2T, l2b,
      Sx, Sh, WA, bA, WB, bB, WC, bC, WD, bD)


def kernel(x, x_real, edge_index, fc_W, fc_b, l1_W, l1_b, l2_W, l2_b, eps,
           convA_W, convA_b, convB_W, convB_b, convC_W, convC_b,
           convD_W, convD_b):
    f32 = jnp.float32

    src = edge_index[0]
    dst = edge_index[1]
    npad = E_PAD - N_EDGES
    ppos = jnp.arange(npad, dtype=jnp.int32)
    src_full = jnp.concatenate([src, ppos % N_NODES])
    dst_full = jnp.concatenate([dst, N_NODES + (ppos % 16)])
    src2d = jnp.stack([src_full, src_full + N_NODES]).reshape(
        2 * E_PAD // 128, 128)
    dst2d = dst_full.reshape(E_PAD // 128, 128)

    x1s = _run_fc(x_real, fc_W, fc_b)

    agg = _make_sc_segment_sum()(
        src2d, dst2d, x1s.reshape(2 * N_NODES, 32))

    l1T = l1_W.T
    l2T = l2_W.T
    ar16 = jnp.arange(16)
    colsx = 8 * (ar16 // 4) + (ar16 % 4)
    Sx = jnp.zeros((16, 32), f32).at[ar16, colsx].set(1.0)
    Sh = jnp.zeros((16, 32), f32).at[ar16, colsx + 4].set(1.0)
    WAf = convA_W.reshape(64, 8)
    WA = jnp.einsum('pq,oj->pjqo', jnp.eye(4, dtype=f32), WAf).reshape(32, 256)
    bA = jnp.tile(convA_b, 4).reshape(1, 256)
    WBf = convB_W[:, :, :, 0].transpose(0, 2, 1).reshape(128, 128)
    WB = jnp.einsum('pq,oj->pjqo', jnp.eye(2, dtype=f32), WBf).reshape(256, 256)
    bB = jnp.tile(convB_b, 2).reshape(1, 256)
    WC = convC_W[:, :, :, 0].transpose(0, 2, 1).reshape(256, 256).T
    bC = convC_b.reshape(1, 256)
    WD = convD_W.reshape(64, 256).T
    bD = convD_b.reshape(1, 64)

    xflat = x.reshape(N_NODES, 16)
    epsp1 = (1.0 + eps).reshape(1, 1)
    out = _run_tail(x1s, agg, xflat, epsp1,
                    l1T, l1_b.reshape(1, 32), l2T, l2_b.reshape(1, 16),
                    Sx, Sh, WA, bA, WB, bB, WC, bC, WD, bD)
    return out.reshape(N_NODES, 64, 1, 1)


# speedup vs baseline: 9.0821x; 1.1532x over previous
import functools

import jax
import jax.numpy as jnp
import numpy as np
from jax import lax
from jax.experimental import pallas as pl
from jax.experimental.pallas import tpu as pltpu
from jax.experimental.pallas import tpu_sc as plsc

N_NODES = 50000
N_EDGES = 800000
NC, NS = 2, 16
EPT = 51200
E_PAD = EPT * NS
W_EDGES = 256
SUPER = 4
NSB = EPT // (SUPER * W_EDGES)
NJ = W_EDGES // 128
NWIN = EPT // W_EDGES
ACC_ROWS = 50048
BN_FC = 5000
BN = 2000


def _leaky(v):
    return jnp.where(v >= 0, v, 0.01 * v)


def _fc_body(xr_ref, wT_ref, b_ref, o_ref):
    y = jnp.dot(xr_ref[...], wT_ref[...], preferred_element_type=jnp.float32)
    y = _leaky(y + b_ref[...])
    o_ref[0] = y[:, :32]
    o_ref[1] = y[:, 32:]


def _run_fc(x_real, fc_W, fc_b):
    grid = N_NODES // BN_FC
    return pl.pallas_call(
        _fc_body,
        grid=(grid,),
        in_specs=[
            pl.BlockSpec((BN_FC, 32), lambda i: (i, 0)),
            pl.BlockSpec((32, 64), lambda i: (0, 0)),
            pl.BlockSpec((1, 64), lambda i: (0, 0)),
        ],
        out_specs=pl.BlockSpec((2, BN_FC, 32), lambda i: (0, i, 0)),
        out_shape=jax.ShapeDtypeStruct((2, N_NODES, 32), jnp.float32),
    )(x_real, fc_W.T, fc_b.reshape(1, 64))


@functools.cache
def _make_sc_segment_sum():
    mesh = plsc.VectorSubcoreMesh(
        core_axis_name="c", subcore_axis_name="s",
        num_cores=NC, num_subcores=NS)
    return pl.kernel(
        _sc_segment_sum_body,
        out_type=jax.ShapeDtypeStruct((NC, ACC_ROWS, 32), jnp.float32),
        mesh=mesh,
        scratch_types=[
            pltpu.VMEM((2, SUPER * NJ, 128), jnp.int32),
            pltpu.VMEM((2, SUPER * NJ, 128), jnp.int32),
            pltpu.VMEM((2, W_EDGES, 32), jnp.float32),
            pltpu.VMEM_SHARED((ACC_ROWS, 32), jnp.float32),
            pltpu.SemaphoreType.DMA,
            pltpu.SemaphoreType.DMA,
            pltpu.SemaphoreType.DMA,
            pltpu.SemaphoreType.DMA,
        ],
        compiler_params=pltpu.CompilerParams(use_tc_tiling_on_sc=False),
    )


def _sc_segment_sum_body(src2d_hbm, dst2d_hbm, x1s_hbm, out_hbm,
                         idx_src, idx_dst, rows, acc,
                         sem_g, sem_s0, sem_s1, sem_i):
    c = lax.axis_index("c")
    s = lax.axis_index("s")
    sem_s = (sem_s0, sem_s1)

    def zbody(i, carry):
        rows[0, i, pl.ds(0, 16)] = jnp.zeros((16,), jnp.float32)
        rows[0, i, pl.ds(16, 16)] = jnp.zeros((16,), jnp.float32)
        return carry

    lax.fori_loop(0, W_EDGES, zbody, 0)
    zrows = ACC_ROWS // NS
    zbase = s * zrows
    for r in range(0, zrows, W_EDGES):
        size = min(W_EDGES, zrows - r)
        pltpu.sync_copy(rows.at[0, pl.ds(0, size)],
                        acc.at[pl.ds(zbase + r, size)])
    plsc.subcore_barrier()

    ept128 = EPT // 128
    src_base = s * ept128
    dst_base = s * ept128
    SBR = SUPER * NJ
    cN = (c * N_NODES).astype(jnp.int32)

    def patch_src(buf):
        for r in range(SBR):
            for l in range(128 // 16):
                idx_src[buf, r, pl.ds(l * 16, 16)] = (
                    idx_src[buf, r, pl.ds(l * 16, 16)] + cN)

    pltpu.sync_copy(src2d_hbm.at[pl.ds(src_base, SBR)], idx_src.at[0])
    pltpu.sync_copy(dst2d_hbm.at[pl.ds(dst_base, SBR)], idx_dst.at[0])
    patch_src(0)

    def superblock(sb, carry):
        ib = sb % 2
        nxt = 1 - ib

        @pl.when(sb > 0)
        def _():
            pltpu.make_async_copy(
                src2d_hbm.at[pl.ds(src_base, SBR)], idx_src.at[ib],
                sem_i).wait()
            pltpu.make_async_copy(
                dst2d_hbm.at[pl.ds(dst_base, SBR)], idx_dst.at[ib],
                sem_i).wait()
            patch_src(ib)

        for k in range(SUPER):
            b = k % 2
            w = sb * SUPER + k

            def drain(b=b):
                for j in range(NJ):
                    pltpu.make_async_copy(
                        rows.at[b, pl.ds(j * 128, 128)],
                        acc.at[idx_dst.at[ib, j]],
                        sem_s[b]).wait()

            if k < 2:
                pl.when(sb > 0)(drain)
            else:
                drain()

            gd = []
            for j in range(NJ):
                gd.append(pltpu.async_copy(
                    x1s_hbm.at[idx_src.at[ib, k * NJ + j]],
                    rows.at[b, pl.ds(j * 128, 128)], sem_g))

            if k == 1:
                @pl.when(sb + 1 < NSB)
                def _():
                    r0 = (sb + 1) * SBR
                    pltpu.async_copy(
                        src2d_hbm.at[pl.ds(src_base + r0, SBR)],
                        idx_src.at[nxt], sem_i)
                    pltpu.async_copy(
                        dst2d_hbm.at[pl.ds(dst_base + r0, SBR)],
                        idx_dst.at[nxt], sem_i)

            for d in gd:
                d.wait()

            for j in range(NJ):
                pltpu.async_copy(
                    rows.at[b, pl.ds(j * 128, 128)],
                    acc.at[idx_dst.at[ib, k * NJ + j]],
                    sem_s[b], add=True)
        return carry

    lax.fori_loop(0, NSB, superblock, 0)

    for b in range(2):
        for j in range(NJ):
            pltpu.make_async_copy(
                rows.at[b, pl.ds(j * 128, 128)],
                acc.at[idx_dst.at[0, j]],
                sem_s[b]).wait()
    plsc.subcore_barrier()

    orows = ACC_ROWS // NS
    pltpu.sync_copy(acc.at[pl.ds(s * orows, orows)],
                    out_hbm.at[c, pl.ds(s * orows, orows)])


def _tail_body(x1_ref, agg_ref, xf_ref, eps_ref,
               l1T_ref, l1b_ref, l2T_ref, l2b_ref,
               sx_ref, sh_ref, wa_ref, ba_ref, wb_ref, bb_ref,
               wc_ref, bc_ref, wd_ref, bd_ref, o_ref):
    f32 = jnp.float32
    x1 = jnp.concatenate([x1_ref[0], x1_ref[1]], axis=1)
    agg = jnp.concatenate([agg_ref[0], agg_ref[1]], axis=1)
    h = eps_ref[0, 0] * x1 + agg
    h = _leaky(jnp.dot(h, l1T_ref[...], preferred_element_type=f32)
               + l1b_ref[...])
    h = _leaky(jnp.dot(h, l2T_ref[...], preferred_element_type=f32)
               + l2b_ref[...])
    h = _leaky(h)
    z = (jnp.dot(xf_ref[...], sx_ref[...], preferred_element_type=f32)
         + jnp.dot(h, sh_ref[...], preferred_element_type=f32))
    ya = _leaky(jnp.dot(z, wa_ref[...], preferred_element_type=f32)
                + ba_ref[...])
    yb = _leaky(jnp.dot(ya, wb_ref[...], preferred_element_type=f32)
                + bb_ref[...])
    yc = _leaky(jnp.dot(yb, wc_ref[...], preferred_element_type=f32)
                + bc_ref[...])
    o_ref[...] = _leaky(jnp.dot(yc, wd_ref[...], preferred_element_type=f32)
                        + bd_ref[...])


def _run_tail(x1s, agg, xflat, epsp1, l1T, l1b, l2T, l2b,
              Sx, Sh, WA, bA, WB, bB, WC, bC, WD, bD):
    grid = N_NODES // BN

    def whole(shape):
        return pl.BlockSpec(shape, lambda i: tuple(0 for _ in shape))

    return pl.pallas_call(
        _tail_body,
        grid=(grid,),
        in_specs=[
            pl.BlockSpec((2, BN, 32), lambda i: (0, i, 0)),
            pl.BlockSpec((2, BN, 32), lambda i: (0, i, 0)),
            pl.BlockSpec((BN, 16), lambda i: (i, 0)),
            whole((1, 1)),
            whole((64, 32)), whole((1, 32)),
            whole((32, 16)), whole((1, 16)),
            whole((16, 32)), whole((16, 32)),
            whole((32, 256)), whole((1, 256)),
            whole((256, 256)), whole((1, 256)),
            whole((256, 256)), whole((1, 256)),
            whole((256, 64)), whole((1, 64)),
        ],
        out_specs=pl.BlockSpec((BN, 64), lambda i: (i, 0)),
        out_shape=jax.ShapeDtypeStruct((N_NODES, 64), jnp.float32),
    )(x1s, agg, xflat, epsp1, l1T, l1b, l2T, l2b,
      Sx, Sh, WA, bA, WB, bB, WC, bC, WD, bD)


def kernel(x, x_real, edge_index, fc_W, fc_b, l1_W, l1_b, l2_W, l2_b, eps,
           convA_W, convA_b, convB_W, convB_b, convC_W, convC_b,
           convD_W, convD_b):
    f32 = jnp.float32

    npad = E_PAD - N_EDGES
    pad_src = jnp.asarray(np.arange(npad, dtype=np.int32) % N_NODES)
    pad_dst = jnp.asarray(
        N_NODES + (np.arange(npad, dtype=np.int32) % 16))
    src2d = jnp.concatenate([edge_index[0], pad_src]).reshape(
        E_PAD // 128, 128)
    dst2d = jnp.concatenate([edge_index[1], pad_dst]).reshape(
        E_PAD // 128, 128)

    x1s = _run_fc(x_real, fc_W, fc_b)

    agg = _make_sc_segment_sum()(
        src2d, dst2d, x1s.reshape(2 * N_NODES, 32))

    l1T = l1_W.T
    l2T = l2_W.T
    ar16 = jnp.arange(16)
    colsx = 8 * (ar16 // 4) + (ar16 % 4)
    Sx = jnp.zeros((16, 32), f32).at[ar16, colsx].set(1.0)
    Sh = jnp.zeros((16, 32), f32).at[ar16, colsx + 4].set(1.0)
    WAf = convA_W.reshape(64, 8)
    WA = jnp.einsum('pq,oj->pjqo', jnp.eye(4, dtype=f32), WAf).reshape(32, 256)
    bA = jnp.tile(convA_b, 4).reshape(1, 256)
    WBf = convB_W[:, :, :, 0].transpose(0, 2, 1).reshape(128, 128)
    WB = jnp.einsum('pq,oj->pjqo', jnp.eye(2, dtype=f32), WBf).reshape(256, 256)
    bB = jnp.tile(convB_b, 2).reshape(1, 256)
    WC = convC_W[:, :, :, 0].transpose(0, 2, 1).reshape(256, 256).T
    bC = convC_b.reshape(1, 256)
    WD = convD_W.reshape(64, 256).T
    bD = convD_b.reshape(1, 64)

    xflat = x.reshape(N_NODES, 16)
    epsp1 = (1.0 + eps).reshape(1, 1)
    out = _run_tail(x1s, agg, xflat, epsp1,
                    l1T, l1_b.reshape(1, 32), l2T, l2_b.reshape(1, 16),
                    Sx, Sh, WA, bA, WB, bB, WC, bC, WD, bD)
    return out.reshape(N_NODES, 64, 1, 1)
